# baseline (speedup 1.0000x reference)
import functools

import jax
import jax.numpy as jnp
from jax import lax
from jax.experimental import pallas as pl
from jax.experimental.pallas import tpu as pltpu
from jax.experimental.pallas import tpu_sc as plsc

N_NODES = 100000
D = 64
PW = 2 * D
NT = D * D
NS = 16
CH = 128
N_FULL = N_NODES // CH
TAIL = N_NODES - N_FULL * CH
N_PAD = (N_FULL + 1) * CH
K_COMMON = N_FULL // NS
N_EXTRA = N_FULL - K_COMMON * NS
NBUF = 4

_mesh = plsc.VectorSubcoreMesh(core_axis_name="c", subcore_axis_name="s")


@functools.partial(
    pl.kernel,
    mesh=_mesh,
    out_type=jax.ShapeDtypeStruct((N_NODES, 2 * PW), jnp.float32),
    scratch_types=[
        pltpu.VMEM_SHARED((NT, PW), jnp.float32),
        pltpu.VMEM((2, CH), jnp.int32),
        pltpu.VMEM((2, CH), jnp.int32),
        pltpu.VMEM((CH, PW), jnp.float32),
        pltpu.VMEM((CH, PW), jnp.float32),
        pltpu.VMEM((CH, PW), jnp.float32),
        pltpu.VMEM((CH, PW), jnp.float32),
        pltpu.SemaphoreType.DMA,
        pltpu.SemaphoreType.DMA,
        pltpu.SemaphoreType.DMA,
        pltpu.SemaphoreType.DMA,
        pltpu.SemaphoreType.DMA,
        pltpu.SemaphoreType.DMA,
    ],
)
def _gather(table_hbm, idx_hbm, out_hbm,
            table_sp, ib0, ib1, b0, b1, b2, b3,
            i0, i1, g0, g1, g2, g3):
    c = lax.axis_index("c")
    s = lax.axis_index("s")

    @pl.when(s == 0)
    def _stage():
        tb = pl.multiple_of(NT * c, NT)
        pltpu.sync_copy(table_hbm.at[pl.ds(tb, NT)], table_sp)

    plsc.subcore_barrier()

    ibs = (ib0, ib1)
    isems = (i0, i1)
    dbs = (b0, b1, b2, b3)
    gsems = (g0, g1, g2, g3)
    colbase = pl.multiple_of(PW * c, PW)

    def off_of(k):
        return pl.multiple_of(CH * NS * k + CH * s, CH)

    def load_idx(k):
        b = k & 1
        return pltpu.async_copy(
            idx_hbm.at[:, pl.ds(off_of(k), CH)], ibs[b], isems[b])

    def gather(k):
        r = k % NBUF
        return pltpu.async_copy(
            table_sp.at[ibs[k & 1].at[c]], dbs[r], gsems[r])

    def store(k):
        r = k % NBUF
        return pltpu.async_copy(
            dbs[r],
            out_hbm.at[pl.ds(off_of(k), CH), pl.ds(colbase, PW)],
            gsems[r])

    n = K_COMMON
    icps = [None] * (n + 1)
    gcps = [None] * n
    scps = [None] * n

    icps[0] = load_idx(0)
    icps[0].wait()
    gcps[0] = gather(0)
    if n > 1:
        icps[1] = load_idx(1)
    for k in range(n):
        gcps[k].wait()
        scps[k] = store(k)
        if k + 1 < n:
            icps[k + 1].wait()
            if k >= NBUF - 1:
                scps[k - (NBUF - 1)].wait()
            gcps[k + 1] = gather(k + 1)
            if k + 2 < n:
                icps[k + 2] = load_idx(k + 2)
    for k in range(max(0, n - NBUF), n):
        scps[k].wait()

    @pl.when(s < N_EXTRA)
    def _extra():
        off = pl.multiple_of(CH * NS * K_COMMON + CH * s, CH)
        pltpu.sync_copy(idx_hbm.at[:, pl.ds(off, CH)], ib0)
        pltpu.async_copy(table_sp.at[ib0.at[c]], b0, g0).wait()
        pltpu.sync_copy(b0, out_hbm.at[pl.ds(off, CH), pl.ds(colbase, PW)])

    @pl.when(s == NS - 1)
    def _tail():
        off = N_FULL * CH
        pltpu.sync_copy(idx_hbm.at[:, pl.ds(off, CH)], ib0)
        pltpu.async_copy(table_sp.at[ib0.at[c]], b0, g0).wait()
        pltpu.sync_copy(
            b0.at[pl.ds(0, TAIL)],
            out_hbm.at[pl.ds(off, TAIL), pl.ds(colbase, PW)])


def kernel(x, W0, W1, W2, W3):
    left = jnp.broadcast_to(
        jnp.stack([W0, W2])[:, :, None, :], (2, D, D, D))
    right = jnp.broadcast_to(
        jnp.stack([W1, W3])[:, None, :, :], (2, D, D, D))
    table = jnp.concatenate([left, right], axis=3).reshape(2 * NT, PW)
    x32 = x.astype(jnp.int32)
    i01 = x32[:, 0] * D + x32[:, 1]
    i23 = x32[:, 2] * D + x32[:, 3]
    idx = jnp.stack([i01, i23])
    idx = jnp.pad(idx, ((0, 0), (0, N_PAD - N_NODES)))
    return _gather(table, idx)


# speedup vs baseline: 1.0152x; 1.0152x over previous
import functools

import jax
import jax.numpy as jnp
from jax import lax
from jax.experimental import pallas as pl
from jax.experimental.pallas import tpu as pltpu
from jax.experimental.pallas import tpu_sc as plsc

N_NODES = 100000
D = 64
PW = 2 * D
NT = D * D
NS = 16
CH = 128
N_FULL = N_NODES // CH
TAIL = N_NODES - N_FULL * CH
N_PAD = (N_FULL + 1) * CH
K_COMMON = N_FULL // NS
N_EXTRA = N_FULL - K_COMMON * NS
NBUF = 4

_mesh = plsc.VectorSubcoreMesh(core_axis_name="c", subcore_axis_name="s")


@functools.partial(
    pl.kernel,
    mesh=_mesh,
    out_type=jax.ShapeDtypeStruct((N_NODES, 2 * PW), jnp.float32),
    scratch_types=[
        pltpu.VMEM_SHARED((NT, PW), jnp.float32),
        pltpu.VMEM((2, CH), jnp.int32),
        pltpu.VMEM((2, CH), jnp.int32),
        pltpu.VMEM((CH, PW), jnp.float32),
        pltpu.VMEM((CH, PW), jnp.float32),
        pltpu.VMEM((CH, PW), jnp.float32),
        pltpu.VMEM((CH, PW), jnp.float32),
        pltpu.SemaphoreType.DMA,
        pltpu.SemaphoreType.DMA,
        pltpu.SemaphoreType.DMA,
        pltpu.SemaphoreType.DMA,
        pltpu.SemaphoreType.DMA,
        pltpu.SemaphoreType.DMA,
    ],
)
def _gather(table_hbm, idx_hbm, out_hbm,
            table_sp, ib0, ib1, b0, b1, b2, b3,
            i0, i1, g0, g1, g2, g3):
    c = lax.axis_index("c")
    s = lax.axis_index("s")

    @pl.when(s == 0)
    def _stage():
        tb = pl.multiple_of(NT * c, NT)
        pltpu.sync_copy(table_hbm.at[pl.ds(tb, NT)], table_sp)

    plsc.subcore_barrier()

    ibs = (ib0, ib1)
    isems = (i0, i1)
    dbs = (b0, b1, b2, b3)
    gsems = (g0, g1, g2, g3)
    colbase = pl.multiple_of(PW * c, PW)

    def off_of(k):
        return pl.multiple_of(CH * NS * k + CH * s, CH)

    def load_idx(k):
        b = k & 1
        return pltpu.async_copy(
            idx_hbm.at[:, pl.ds(off_of(k), CH)], ibs[b], isems[b])

    def gather(k):
        r = k % NBUF
        return pltpu.async_copy(
            table_sp.at[ibs[k & 1].at[c]], dbs[r], gsems[r])

    def store(k):
        r = k % NBUF
        return pltpu.async_copy(
            dbs[r],
            out_hbm.at[pl.ds(off_of(k), CH), pl.ds(colbase, PW)],
            gsems[r])

    n = K_COMMON
    icps = [None] * (n + 1)
    gcps = [None] * n
    scps = [None] * n

    icps[0] = load_idx(0)
    icps[0].wait()
    gcps[0] = gather(0)
    if n > 1:
        icps[1] = load_idx(1)
    for k in range(n):
        gcps[k].wait()
        scps[k] = store(k)
        if k + 1 < n:
            icps[k + 1].wait()
            if k >= NBUF - 1:
                scps[k - (NBUF - 1)].wait()
            gcps[k + 1] = gather(k + 1)
            if k + 2 < n:
                icps[k + 2] = load_idx(k + 2)
    for k in range(max(0, n - NBUF), n):
        scps[k].wait()

    @pl.when(s < N_EXTRA)
    def _extra():
        off = pl.multiple_of(CH * NS * K_COMMON + CH * s, CH)
        pltpu.sync_copy(idx_hbm.at[:, pl.ds(off, CH)], ib0)
        pltpu.async_copy(table_sp.at[ib0.at[c]], b0, g0).wait()
        pltpu.sync_copy(b0, out_hbm.at[pl.ds(off, CH), pl.ds(colbase, PW)])

    @pl.when(s == NS - 1)
    def _tail():
        off = N_FULL * CH
        pltpu.sync_copy(idx_hbm.at[:, pl.ds(off, CH)], ib0)
        pltpu.async_copy(table_sp.at[ib0.at[c]], b0, g0).wait()
        pltpu.sync_copy(
            b0.at[pl.ds(0, TAIL)],
            out_hbm.at[pl.ds(off, TAIL), pl.ds(colbase, PW)])


def kernel(x, W0, W1, W2, W3):
    t01 = jnp.concatenate(
        [jnp.repeat(W0, D, axis=0), jnp.tile(W1, (D, 1))], axis=1)
    t23 = jnp.concatenate(
        [jnp.repeat(W2, D, axis=0), jnp.tile(W3, (D, 1))], axis=1)
    table = jnp.concatenate([t01, t23], axis=0)
    x32 = x.astype(jnp.int32)
    i01 = x32[:, 0] * D + x32[:, 1]
    i23 = x32[:, 2] * D + x32[:, 3]
    idx = jnp.stack([i01, i23])
    idx = jnp.pad(idx, ((0, 0), (0, N_PAD - N_NODES)))
    return _gather(table, idx)


# speedup vs baseline: 1.0185x; 1.0032x over previous
import functools

import jax
import jax.numpy as jnp
from jax import lax
from jax.experimental import pallas as pl
from jax.experimental.pallas import tpu as pltpu
from jax.experimental.pallas import tpu_sc as plsc

N_NODES = 100000
D = 64
PW = 2 * D
NT = D * D
NS = 16
CH = 128
N_FULL = N_NODES // CH
TAIL = N_NODES - N_FULL * CH
N_PAD = (N_FULL + 1) * CH
K_COMMON = N_FULL // NS
N_EXTRA = N_FULL - K_COMMON * NS
NBUF = 5

_mesh = plsc.VectorSubcoreMesh(core_axis_name="c", subcore_axis_name="s")


@functools.partial(
    pl.kernel,
    mesh=_mesh,
    out_type=jax.ShapeDtypeStruct((N_NODES, 2 * PW), jnp.float32),
    scratch_types=[
        pltpu.VMEM_SHARED((NT, PW), jnp.float32),
        pltpu.VMEM((2, CH), jnp.int32),
        pltpu.VMEM((2, CH), jnp.int32),
        pltpu.VMEM((CH, PW), jnp.float32),
        pltpu.VMEM((CH, PW), jnp.float32),
        pltpu.VMEM((CH, PW), jnp.float32),
        pltpu.VMEM((CH, PW), jnp.float32),
        pltpu.VMEM((CH, PW), jnp.float32),
        pltpu.SemaphoreType.DMA,
        pltpu.SemaphoreType.DMA,
        pltpu.SemaphoreType.DMA,
        pltpu.SemaphoreType.DMA,
        pltpu.SemaphoreType.DMA,
        pltpu.SemaphoreType.DMA,
        pltpu.SemaphoreType.DMA,
    ],
)
def _gather(table_hbm, idx_hbm, out_hbm,
            table_sp, ib0, ib1, b0, b1, b2, b3, b4,
            i0, i1, g0, g1, g2, g3, g4):
    c = lax.axis_index("c")
    s = lax.axis_index("s")

    @pl.when(s == 0)
    def _stage():
        tb = pl.multiple_of(NT * c, NT)
        pltpu.sync_copy(table_hbm.at[pl.ds(tb, NT)], table_sp)

    plsc.subcore_barrier()

    ibs = (ib0, ib1)
    isems = (i0, i1)
    dbs = (b0, b1, b2, b3, b4)
    gsems = (g0, g1, g2, g3, g4)
    colbase = pl.multiple_of(PW * c, PW)

    def off_of(k):
        return pl.multiple_of(CH * NS * k + CH * s, CH)

    def load_idx(k):
        b = k & 1
        return pltpu.async_copy(
            idx_hbm.at[:, pl.ds(off_of(k), CH)], ibs[b], isems[b])

    def gather(k):
        r = k % NBUF
        return pltpu.async_copy(
            table_sp.at[ibs[k & 1].at[c]], dbs[r], gsems[r])

    def store(k):
        r = k % NBUF
        return pltpu.async_copy(
            dbs[r],
            out_hbm.at[pl.ds(off_of(k), CH), pl.ds(colbase, PW)],
            gsems[r])

    n = K_COMMON
    icps = [None] * (n + 1)
    gcps = [None] * n
    scps = [None] * n

    icps[0] = load_idx(0)
    icps[0].wait()
    gcps[0] = gather(0)
    if n > 1:
        icps[1] = load_idx(1)
    for k in range(n):
        gcps[k].wait()
        scps[k] = store(k)
        if k + 1 < n:
            icps[k + 1].wait()
            if k >= NBUF - 1:
                scps[k - (NBUF - 1)].wait()
            gcps[k + 1] = gather(k + 1)
            if k + 2 < n:
                icps[k + 2] = load_idx(k + 2)
    for k in range(max(0, n - NBUF), n):
        scps[k].wait()

    @pl.when(s < N_EXTRA)
    def _extra():
        off = pl.multiple_of(CH * NS * K_COMMON + CH * s, CH)
        pltpu.sync_copy(idx_hbm.at[:, pl.ds(off, CH)], ib0)
        pltpu.async_copy(table_sp.at[ib0.at[c]], b0, g0).wait()
        pltpu.sync_copy(b0, out_hbm.at[pl.ds(off, CH), pl.ds(colbase, PW)])

    @pl.when(s == NS - 1)
    def _tail():
        off = N_FULL * CH
        pltpu.sync_copy(idx_hbm.at[:, pl.ds(off, CH)], ib0)
        pltpu.async_copy(table_sp.at[ib0.at[c]], b0, g0).wait()
        pltpu.sync_copy(
            b0.at[pl.ds(0, TAIL)],
            out_hbm.at[pl.ds(off, TAIL), pl.ds(colbase, PW)])


def kernel(x, W0, W1, W2, W3):
    t01 = jnp.concatenate(
        [jnp.repeat(W0, D, axis=0), jnp.tile(W1, (D, 1))], axis=1)
    t23 = jnp.concatenate(
        [jnp.repeat(W2, D, axis=0), jnp.tile(W3, (D, 1))], axis=1)
    table = jnp.concatenate([t01, t23], axis=0)
    x32 = x.astype(jnp.int32)
    i01 = x32[:, 0] * D + x32[:, 1]
    i23 = x32[:, 2] * D + x32[:, 3]
    idx = jnp.stack([i01, i23])
    idx = jnp.pad(idx, ((0, 0), (0, N_PAD - N_NODES)))
    return _gather(table, idx)
